# baseline (speedup 1.0000x reference)
import functools

import jax
import jax.numpy as jnp
from jax import lax
from jax.experimental import pallas as pl
from jax.experimental.pallas import tpu as pltpu
from jax.experimental.pallas import tpu_sc as plsc

N = 10000
E = 320000
HID = 128
LBL = 64

NC, NS = 2, 16
NW = NC * NS
CH = 128
EW = 10240
E_PAD = NW * EW
NCHUNK = EW // CH
GCH = 64
GNCH = EW // GCH
A_ROWS = 10240
RPT = A_ROWS // NS
NPT = 10016

_MESH = dict(core_axis_name="c", subcore_axis_name="s",
             num_cores=NC, num_subcores=NS)
_SC_PARAMS = pltpu.CompilerParams(needs_layout_passes=False,
                                  use_tc_tiling_on_sc=False)

_f32 = jnp.float32
_i32 = jnp.int32


def _fill(ref, n_rows, n_cols, val):
    v = jnp.full((16,), val, _f32)

    def body(i, _):
        for r in range(n_cols // 16):
            ref[i, pl.ds(r * 16, 16)] = v
        return 0

    lax.fori_loop(0, n_rows, body, 0)


def _zero_acc_slice(zbuf, acc, s):
    def body(k, _):
        pltpu.sync_copy(zbuf, acc.at[pl.ds(s * RPT + k * 16, 16)])
        return 0

    lax.fori_loop(0, RPT // 16, body, 0)


def _row_pass(table, src2d, dst2d, d, ch, seg, nseg0, nseg1, wgt2d=None):
    NB = 4
    assert seg % NB == 0
    n0, n1 = seg * nseg0, seg * nseg1
    assert NS * (n0 + n1) == E_PAD // ch
    weighted = wgt2d is not None

    scratch = [
        pltpu.VMEM((seg, ch), _i32),
        pltpu.VMEM((seg, ch), _i32),
        pltpu.VMEM((NB, ch, d), _f32),
        pltpu.VMEM((16, d), _f32),
        pltpu.VMEM_SHARED((A_ROWS, d), _f32),
        pltpu.SemaphoreType.DMA((NB,)),
    ]
    if weighted:
        scratch.insert(2, pltpu.VMEM((seg, ch), _f32))

    @functools.partial(
        pl.kernel,
        out_type=jax.ShapeDtypeStruct((NC, N, d), _f32),
        mesh=plsc.VectorSubcoreMesh(**_MESH),
        compiler_params=_SC_PARAMS,
        scratch_types=scratch,
    )
    def body(*refs):
        if weighted:
            (tab, src, dst, wgt, out,
             idx_sb, idx_db, wb, rows, zbuf, acc, gsem) = refs
        else:
            (tab, src, dst, out,
             idx_sb, idx_db, rows, zbuf, acc, gsem) = refs
        c = lax.axis_index("c")
        s = lax.axis_index("s")
        _fill(zbuf, 16, d, 0.0)
        _zero_acc_slice(zbuf, acc, s)
        plsc.subcore_barrier()
        nseg_c = jnp.where(c == 0, nseg0, nseg1)
        tbase = jnp.where(c == 0, s * n0, NS * n0 + s * n1)

        for h in range(max(nseg0, nseg1)):
            @pl.when(h < nseg_c)
            def _():
                srow = tbase + h * seg
                pltpu.sync_copy(src.at[pl.ds(srow, seg)], idx_sb)
                pltpu.sync_copy(dst.at[pl.ds(srow, seg)], idx_db)
                if weighted:
                    pltpu.sync_copy(wgt.at[pl.ds(srow, seg)], wb)
                for p in range(NB - 1):
                    pltpu.async_copy(
                        tab.at[idx_sb.at[p]], rows.at[p], gsem.at[p])

                def chunk(i, _):
                    b = lax.rem(i, NB)
                    pltpu.make_async_copy(
                        tab.at[idx_sb.at[i]], rows.at[b], gsem.at[b]).wait()
                    nxt = i + NB - 1
                    nb_ = lax.rem(nxt, NB)

                    @pl.when(nxt < seg)
                    def _():
                        pltpu.async_copy(
                            tab.at[idx_sb.at[nxt]], rows.at[nb_],
                            gsem.at[nb_])

                    if weighted:
                        for g2 in range(ch // 16):
                            w16 = wb[i, pl.ds(g2 * 16, 16)]
                            for j in range(16):
                                w = w16[j]
                                row = g2 * 16 + j
                                for r in range(d // 16):
                                    rows[b, row, pl.ds(r * 16, 16)] = (
                                        rows[b, row, pl.ds(r * 16, 16)] * w)
                    pltpu.sync_copy(rows.at[b], acc.at[idx_db.at[i]], add=True)
                    return 0

                lax.fori_loop(0, seg, chunk, 0)

        plsc.subcore_barrier()

        @pl.when(s == 0)
        def _():
            pltpu.sync_copy(acc.at[pl.ds(0, N)], out.at[c])

    if weighted:
        return body(table, src2d, dst2d, wgt2d)
    return body(table, src2d, dst2d)


def _deg_pass(dstp):
    @functools.partial(
        pl.kernel,
        out_type=jax.ShapeDtypeStruct((NC, N, 16), _f32),
        mesh=plsc.VectorSubcoreMesh(**_MESH),
        compiler_params=_SC_PARAMS,
        scratch_types=[
            pltpu.VMEM((CH,), _i32),
            pltpu.VMEM((CH, 16), _f32),
            pltpu.VMEM((16, 16), _f32),
            pltpu.VMEM_SHARED((A_ROWS, 16), _f32),
        ],
    )
    def body(dst, out, idx_d, crow, zbuf, acc):
        c = lax.axis_index("c")
        s = lax.axis_index("s")
        _fill(zbuf, 16, 16, 0.0)
        _fill(crow, CH, 16, 1.0)
        _zero_acc_slice(zbuf, acc, s)
        plsc.subcore_barrier()
        base0 = (c * NS + s) * EW

        def chunk(i, _):
            base = base0 + i * CH
            pltpu.sync_copy(dst.at[pl.ds(base, CH)], idx_d)
            pltpu.sync_copy(crow, acc.at[idx_d], add=True)
            return 0

        lax.fori_loop(0, NCHUNK, chunk, 0)
        plsc.subcore_barrier()

        @pl.when(s == 0)
        def _():
            pltpu.sync_copy(acc.at[pl.ds(0, N)], out.at[c])

    return body(dstp)


def _wgt_pass(src2d, dst2d, asv, adv, mvec):
    @functools.partial(
        pl.kernel,
        out_type=(
            jax.ShapeDtypeStruct((E_PAD // CH, CH), _f32),
            jax.ShapeDtypeStruct((NC, N, 16), _f32),
        ),
        mesh=plsc.VectorSubcoreMesh(**_MESH),
        compiler_params=_SC_PARAMS,
        scratch_types=[
            pltpu.VMEM((NCHUNK, CH), _i32),
            pltpu.VMEM((NCHUNK, CH), _i32),
            pltpu.VMEM((NCHUNK, CH), _f32),
            pltpu.VMEM((CH, 16), _f32),
            pltpu.VMEM((NPT,), _f32),
            pltpu.VMEM((NPT,), _f32),
            pltpu.VMEM((16,), _f32),
            pltpu.VMEM((16, 16), _f32),
            pltpu.VMEM_SHARED((A_ROWS, 16), _f32),
        ],
    )
    def body(src, dst, a_s, a_d, mv, wout, dout,
             idx_sb, idx_db, wb, ex16, asb, adb, mvb, zb16, den):
        c = lax.axis_index("c")
        s = lax.axis_index("s")
        wrow = (c * NS + s) * NCHUNK
        pltpu.sync_copy(a_s, asb)
        pltpu.sync_copy(a_d, adb)
        pltpu.sync_copy(mv, mvb)
        pltpu.sync_copy(src.at[pl.ds(wrow, NCHUNK)], idx_sb)
        pltpu.sync_copy(dst.at[pl.ds(wrow, NCHUNK)], idx_db)
        _fill(zb16, 16, 16, 0.0)
        _fill(ex16, CH, 16, 0.0)
        _zero_acc_slice(zb16, den, s)
        plsc.subcore_barrier()
        mvv = mvb[pl.ds(0, 16)]
        czero = jnp.zeros((16,), _i32)
        ar16 = jnp.arange(16, dtype=_i32)

        def chunk(i, _):
            for g in range(CH // 16):
                s16 = idx_sb[i, pl.ds(g * 16, 16)]
                d16 = idx_db[i, pl.ds(g * 16, 16)]
                e = plsc.load_gather(asb, [s16]) + plsc.load_gather(adb, [d16])
                e = jnp.where(e >= 0.0, e, e * 0.2)
                ex = jnp.exp(e - mvv)
                wb[i, pl.ds(g * 16, 16)] = ex
                plsc.store_scatter(ex16, [g * 16 + ar16, czero], ex)
            pltpu.sync_copy(ex16, den.at[idx_db.at[i]], add=True)
            return 0

        lax.fori_loop(0, NCHUNK, chunk, 0)
        pltpu.sync_copy(wb, wout.at[pl.ds(wrow, NCHUNK)])
        plsc.subcore_barrier()

        @pl.when(s == 0)
        def _():
            pltpu.sync_copy(den.at[pl.ds(0, N)], dout.at[c])

    return body(src2d, dst2d, asv, adv, mvec)


def _tck1(x_ref, w1_ref, degp_ref, h1hat_ref, dinv_ref):
    x = x_ref[...]
    mu = jnp.mean(x, axis=0, keepdims=True)
    xc = x - mu
    sd = jnp.sqrt(jnp.sum(xc * xc, axis=0, keepdims=True) * (1.0 / (N - 1)))
    xn = xc / sd
    h1p = jnp.dot(xn, w1_ref[...], preferred_element_type=_f32)
    dp = degp_ref[...]
    deg = dp[0, :, 0:1] + dp[1, :, 0:1] + 1.0
    dinv = lax.rsqrt(deg)
    dinv_ref[...] = dinv
    h1hat_ref[...] = dinv * h1p


def _tck2(accp_ref, h1hat_ref, dinv_ref, wg_ref, atts_ref, attd_ref,
          b1_ref, h2p_ref, as_ref, ad_ref, m_ref):
    a = accp_ref[...]
    dinv = dinv_ref[...]
    out1 = dinv * (a[0] + a[1] + h1hat_ref[...]) + b1_ref[...]
    h1 = jnp.maximum(out1, 0.0)
    h2p = jnp.dot(h1, wg_ref[...], preferred_element_type=_f32)
    a_s = jnp.dot(h2p, atts_ref[...], preferred_element_type=_f32)
    a_d = jnp.dot(h2p, attd_ref[...], preferred_element_type=_f32)
    m = jnp.max(a_s) + jnp.max(a_d)
    m = jnp.where(m >= 0.0, m, m * 0.2)
    h2p_ref[...] = h2p
    as_ref[...] = a_s
    ad_ref[...] = a_d
    m_ref[...] = jnp.reshape(m, (1, 1))


def _tck3(nump_ref, denp_ref, as_ref, ad_ref, m_ref, h2p_ref, dinv_ref,
          w2_ref, bg_ref, h3hat_ref):
    m = m_ref[0, 0]
    es = as_ref[...] + ad_ref[...]
    es = jnp.where(es >= 0.0, es, es * 0.2)
    exs = jnp.exp(es - m)
    npv = nump_ref[...]
    h2p = h2p_ref[...]
    num = npv[0] + npv[1] + exs * h2p
    dp = denp_ref[...]
    den = dp[0, :, 0:1] + dp[1, :, 0:1] + exs
    h2 = jnp.maximum(num / (den + 1e-16) + bg_ref[...], 0.0)
    h3p = jnp.dot(h2, w2_ref[...], preferred_element_type=_f32)
    h3hat_ref[...] = dinv_ref[...] * h3p


def _tck4(accp_ref, h3hat_ref, dinv_ref, b2_ref, y_ref):
    a = accp_ref[...]
    out3 = dinv_ref[...] * (a[0] + a[1] + h3hat_ref[...]) + b2_ref[...]
    mx = jnp.max(out3, axis=1, keepdims=True)
    lse = mx + jnp.log(jnp.sum(jnp.exp(out3 - mx), axis=1, keepdims=True))
    y_ref[...] = out3 - lse


def _tc(body, out_shape, *args):
    return pl.pallas_call(body, out_shape=out_shape)(*args)


def kernel(x, edge_index, W1, b1, Wg, att_src, att_dst, bg, W2, b2):
    src = edge_index[0].astype(_i32)
    dst = edge_index[1].astype(_i32)
    pad = E_PAD - E
    srcp = jnp.concatenate([src, jnp.zeros((pad,), _i32)])
    dstp = jnp.concatenate([dst, jnp.full((pad,), N, _i32)])
    src2d = srcp.reshape(E_PAD // CH, CH)
    dst2d = dstp.reshape(E_PAD // CH, CH)
    src2g = srcp.reshape(E_PAD // GCH, GCH)
    dst2g = dstp.reshape(E_PAD // GCH, GCH)

    degp = _deg_pass(dstp)

    h1hat, dinv = _tc(
        _tck1,
        (jax.ShapeDtypeStruct((N, HID), _f32),
         jax.ShapeDtypeStruct((N, 1), _f32)),
        x, W1, degp)

    acc1 = _row_pass(h1hat, src2g, dst2g, HID, GCH, 32, 10, 0)

    h2p, a_s, a_d, mval = _tc(
        _tck2,
        (jax.ShapeDtypeStruct((N, HID), _f32),
         jax.ShapeDtypeStruct((N, 1), _f32),
         jax.ShapeDtypeStruct((N, 1), _f32),
         jax.ShapeDtypeStruct((1, 1), _f32)),
        acc1, h1hat, dinv, Wg, att_src.reshape(HID, 1),
        att_dst.reshape(HID, 1), b1.reshape(1, HID))

    asv = jnp.pad(a_s.reshape(-1), (0, NPT - N))
    adv = jnp.pad(a_d.reshape(-1), (0, NPT - N))
    mvec = jnp.broadcast_to(mval.reshape(1), (16,))

    wgt, denp = _wgt_pass(src2d, dst2d, asv, adv, mvec)
    w2g = wgt.reshape(E_PAD // GCH, GCH)

    nump = _row_pass(h2p, src2g, dst2g, HID, GCH, 40, 8, 0, wgt2d=w2g)

    h3hat = _tc(
        _tck3,
        jax.ShapeDtypeStruct((N, LBL), _f32),
        nump, denp, a_s, a_d, mval, h2p, dinv, W2, bg.reshape(1, HID))

    acc3 = _row_pass(h3hat, src2d, dst2d, LBL, CH, 16, 9, 1)

    y = _tc(
        _tck4,
        jax.ShapeDtypeStruct((N, LBL), _f32),
        acc3, h3hat, dinv, b2.reshape(1, LBL))
    return y


# speedup vs baseline: 1.3184x; 1.3184x over previous
import functools

import jax
import jax.numpy as jnp
from jax import lax
from jax.experimental import pallas as pl
from jax.experimental.pallas import tpu as pltpu
from jax.experimental.pallas import tpu_sc as plsc

N = 10000
E = 320000
HID = 128
LBL = 64

NC, NS = 2, 16
NW = NC * NS
CH = 128
EW = 10240
E_PAD = NW * EW
NCHUNK = EW // CH
GCH = 64
GNCH = EW // GCH
A_ROWS = 10240
RPT = A_ROWS // NS
NPT = 10016

_MESH = dict(core_axis_name="c", subcore_axis_name="s",
             num_cores=NC, num_subcores=NS)
_SC_PARAMS = pltpu.CompilerParams(needs_layout_passes=False,
                                  use_tc_tiling_on_sc=False)

_f32 = jnp.float32
_i32 = jnp.int32


def _fill(ref, n_rows, n_cols, val):
    v = jnp.full((16,), val, _f32)

    def body(i, _):
        for r in range(n_cols // 16):
            ref[i, pl.ds(r * 16, 16)] = v
        return 0

    lax.fori_loop(0, n_rows, body, 0)


def _zero_acc_slice(zbuf, acc, s):
    def body(k, _):
        pltpu.sync_copy(zbuf, acc.at[pl.ds(s * RPT + k * 16, 16)])
        return 0

    lax.fori_loop(0, RPT // 16, body, 0)


def _row_pass(table, src2d, dst2d, d, ch, seg, nseg0, nseg1, wgt2d=None):
    NB = 4
    assert seg % NB == 0
    n0, n1 = seg * nseg0, seg * nseg1
    assert NS * (n0 + n1) == E_PAD // ch
    weighted = wgt2d is not None

    scratch = [
        pltpu.VMEM((seg, ch), _i32),
        pltpu.VMEM((seg, ch), _i32),
        pltpu.VMEM((NB, ch, d), _f32),
        pltpu.VMEM((16, d), _f32),
        pltpu.VMEM_SHARED((A_ROWS, d), _f32),
        pltpu.SemaphoreType.DMA((NB,)),
    ]
    if weighted:
        scratch.insert(2, pltpu.VMEM((seg, ch), _f32))

    @functools.partial(
        pl.kernel,
        out_type=jax.ShapeDtypeStruct((NC, N, d), _f32),
        mesh=plsc.VectorSubcoreMesh(**_MESH),
        compiler_params=_SC_PARAMS,
        scratch_types=scratch,
    )
    def body(*refs):
        if weighted:
            (tab, src, dst, wgt, out,
             idx_sb, idx_db, wb, rows, zbuf, acc, gsem) = refs
        else:
            (tab, src, dst, out,
             idx_sb, idx_db, rows, zbuf, acc, gsem) = refs
        c = lax.axis_index("c")
        s = lax.axis_index("s")
        _fill(zbuf, 16, d, 0.0)
        _zero_acc_slice(zbuf, acc, s)
        plsc.subcore_barrier()
        nseg_c = jnp.where(c == 0, nseg0, nseg1)
        tbase = jnp.where(c == 0, s * n0, NS * n0 + s * n1)

        for h in range(max(nseg0, nseg1)):
            @pl.when(h < nseg_c)
            def _():
                srow = tbase + h * seg
                pltpu.sync_copy(src.at[pl.ds(srow, seg)], idx_sb)
                pltpu.sync_copy(dst.at[pl.ds(srow, seg)], idx_db)
                if weighted:
                    pltpu.sync_copy(wgt.at[pl.ds(srow, seg)], wb)
                for p in range(NB - 1):
                    pltpu.async_copy(
                        tab.at[idx_sb.at[p]], rows.at[p], gsem.at[p])

                def chunk(i, _):
                    b = lax.rem(i, NB)
                    pltpu.make_async_copy(
                        tab.at[idx_sb.at[i]], rows.at[b], gsem.at[b]).wait()
                    nxt = i + NB - 1
                    nb_ = lax.rem(nxt, NB)

                    @pl.when(nxt < seg)
                    def _():
                        pltpu.async_copy(
                            tab.at[idx_sb.at[nxt]], rows.at[nb_],
                            gsem.at[nb_])

                    if weighted:
                        for g2 in range(ch // 16):
                            w16 = wb[i, pl.ds(g2 * 16, 16)]
                            for j in range(16):
                                w = w16[j]
                                row = g2 * 16 + j
                                for r in range(d // 16):
                                    rows[b, row, pl.ds(r * 16, 16)] = (
                                        rows[b, row, pl.ds(r * 16, 16)] * w)
                    pltpu.sync_copy(rows.at[b], acc.at[idx_db.at[i]], add=True)
                    return 0

                lax.fori_loop(0, seg, chunk, 0)

        plsc.subcore_barrier()

        @pl.when(s == 0)
        def _():
            pltpu.sync_copy(acc.at[pl.ds(0, N)], out.at[c])

    if weighted:
        return body(table, src2d, dst2d, wgt2d)
    return body(table, src2d, dst2d)


def _deg_pass(dstp):
    @functools.partial(
        pl.kernel,
        out_type=jax.ShapeDtypeStruct((NC, N, 16), _f32),
        mesh=plsc.VectorSubcoreMesh(**_MESH),
        compiler_params=_SC_PARAMS,
        scratch_types=[
            pltpu.VMEM((CH,), _i32),
            pltpu.VMEM((CH, 16), _f32),
            pltpu.VMEM((16, 16), _f32),
            pltpu.VMEM_SHARED((A_ROWS, 16), _f32),
        ],
    )
    def body(dst, out, idx_d, crow, zbuf, acc):
        c = lax.axis_index("c")
        s = lax.axis_index("s")
        _fill(zbuf, 16, 16, 0.0)
        _fill(crow, CH, 16, 1.0)
        _zero_acc_slice(zbuf, acc, s)
        plsc.subcore_barrier()
        base0 = (c * NS + s) * EW

        def chunk(i, _):
            base = base0 + i * CH
            pltpu.sync_copy(dst.at[pl.ds(base, CH)], idx_d)
            pltpu.sync_copy(crow, acc.at[idx_d], add=True)
            return 0

        lax.fori_loop(0, NCHUNK, chunk, 0)
        plsc.subcore_barrier()

        @pl.when(s == 0)
        def _():
            pltpu.sync_copy(acc.at[pl.ds(0, N)], out.at[c])

    return body(dstp)


def _wgt_pass(src2d, dst2d, asv, adv, mvec):
    @functools.partial(
        pl.kernel,
        out_type=(
            jax.ShapeDtypeStruct((E_PAD // CH, CH), _f32),
            jax.ShapeDtypeStruct((NC, N, 16), _f32),
        ),
        mesh=plsc.VectorSubcoreMesh(**_MESH),
        compiler_params=_SC_PARAMS,
        scratch_types=[
            pltpu.VMEM((NCHUNK, CH), _i32),
            pltpu.VMEM((NCHUNK, CH), _i32),
            pltpu.VMEM((NCHUNK, CH), _f32),
            pltpu.VMEM((CH, 16), _f32),
            pltpu.VMEM((NPT,), _f32),
            pltpu.VMEM((NPT,), _f32),
            pltpu.VMEM((16,), _f32),
            pltpu.VMEM((16, 16), _f32),
            pltpu.VMEM_SHARED((A_ROWS, 16), _f32),
        ],
    )
    def body(src, dst, a_s, a_d, mv, wout, dout,
             idx_sb, idx_db, wb, ex16, asb, adb, mvb, zb16, den):
        c = lax.axis_index("c")
        s = lax.axis_index("s")
        wrow = (c * NS + s) * NCHUNK
        pltpu.sync_copy(a_s, asb)
        pltpu.sync_copy(a_d, adb)
        pltpu.sync_copy(mv, mvb)
        pltpu.sync_copy(src.at[pl.ds(wrow, NCHUNK)], idx_sb)
        pltpu.sync_copy(dst.at[pl.ds(wrow, NCHUNK)], idx_db)
        _fill(zb16, 16, 16, 0.0)
        _fill(ex16, CH, 16, 0.0)
        _zero_acc_slice(zb16, den, s)
        plsc.subcore_barrier()
        mvv = mvb[pl.ds(0, 16)]
        czero = jnp.zeros((16,), _i32)
        ar16 = jnp.arange(16, dtype=_i32)

        def chunk(i, _):
            for g in range(CH // 16):
                s16 = idx_sb[i, pl.ds(g * 16, 16)]
                d16 = idx_db[i, pl.ds(g * 16, 16)]
                e = plsc.load_gather(asb, [s16]) + plsc.load_gather(adb, [d16])
                e = jnp.where(e >= 0.0, e, e * 0.2)
                ex = jnp.exp(e - mvv)
                wb[i, pl.ds(g * 16, 16)] = ex
                plsc.store_scatter(ex16, [g * 16 + ar16, czero], ex)
            pltpu.sync_copy(ex16, den.at[idx_db.at[i]], add=True)
            return 0

        lax.fori_loop(0, NCHUNK, chunk, 0)
        pltpu.sync_copy(wb, wout.at[pl.ds(wrow, NCHUNK)])
        plsc.subcore_barrier()

        @pl.when(s == 0)
        def _():
            pltpu.sync_copy(den.at[pl.ds(0, N)], dout.at[c])

    return body(src2d, dst2d, asv, adv, mvec)


def _tck1(x_ref, w1_ref, degp_ref, h1hat_ref, dinv_ref):
    x = x_ref[...]
    mu = jnp.mean(x, axis=0, keepdims=True)
    xc = x - mu
    sd = jnp.sqrt(jnp.sum(xc * xc, axis=0, keepdims=True) * (1.0 / (N - 1)))
    xn = xc / sd
    h1p = jnp.dot(xn, w1_ref[...], preferred_element_type=_f32)
    dp = degp_ref[...]
    deg = dp[0, :, 0:1] + dp[1, :, 0:1] + 1.0
    dinv = lax.rsqrt(deg)
    dinv_ref[...] = dinv
    h1hat_ref[...] = dinv * h1p


def _tck2(accp_ref, h1hat_ref, dinv_ref, wg_ref, atts_ref, attd_ref,
          b1_ref, h2p_ref, as_ref, ad_ref, m_ref):
    a = accp_ref[...]
    dinv = dinv_ref[...]
    out1 = dinv * (a[0] + a[1] + h1hat_ref[...]) + b1_ref[...]
    h1 = jnp.maximum(out1, 0.0)
    h2p = jnp.dot(h1, wg_ref[...], preferred_element_type=_f32)
    a_s = jnp.dot(h2p, atts_ref[...], preferred_element_type=_f32)
    a_d = jnp.dot(h2p, attd_ref[...], preferred_element_type=_f32)
    m = jnp.max(a_s) + jnp.max(a_d)
    m = jnp.where(m >= 0.0, m, m * 0.2)
    h2p_ref[...] = h2p
    as_ref[...] = a_s
    ad_ref[...] = a_d
    m_ref[...] = jnp.reshape(m, (1, 1))


def _tck3(nump_ref, denp_ref, as_ref, ad_ref, m_ref, h2p_ref, dinv_ref,
          w2_ref, bg_ref, h3hat_ref):
    m = m_ref[0, 0]
    es = as_ref[...] + ad_ref[...]
    es = jnp.where(es >= 0.0, es, es * 0.2)
    exs = jnp.exp(es - m)
    npv = nump_ref[...]
    h2p = h2p_ref[...]
    num = npv[0] + npv[1] + exs * h2p
    dp = denp_ref[...]
    den = dp[0, :, 0:1] + dp[1, :, 0:1] + exs
    h2 = jnp.maximum(num / (den + 1e-16) + bg_ref[...], 0.0)
    h3p = jnp.dot(h2, w2_ref[...], preferred_element_type=_f32)
    h3hat_ref[...] = dinv_ref[...] * h3p


def _tck4(accp_ref, h3hat_ref, dinv_ref, b2_ref, y_ref):
    a = accp_ref[...]
    out3 = dinv_ref[...] * (a[0] + a[1] + h3hat_ref[...]) + b2_ref[...]
    mx = jnp.max(out3, axis=1, keepdims=True)
    lse = mx + jnp.log(jnp.sum(jnp.exp(out3 - mx), axis=1, keepdims=True))
    y_ref[...] = out3 - lse


def _tc(body, out_shape, *args):
    return pl.pallas_call(body, out_shape=out_shape)(*args)


def kernel(x, edge_index, W1, b1, Wg, att_src, att_dst, bg, W2, b2):
    src = edge_index[0].astype(_i32)
    dst = edge_index[1].astype(_i32)
    pad = E_PAD - E
    srcp = jnp.concatenate([src, jnp.zeros((pad,), _i32)])
    dstp = jnp.concatenate([dst, jnp.full((pad,), N, _i32)])
    src2d = srcp.reshape(E_PAD // CH, CH)
    dst2d = dstp.reshape(E_PAD // CH, CH)
    src2g = srcp.reshape(E_PAD // GCH, GCH)
    dst2g = dstp.reshape(E_PAD // GCH, GCH)

    degp = _deg_pass(dstp)

    h1hat, dinv = _tc(
        _tck1,
        (jax.ShapeDtypeStruct((N, HID), _f32),
         jax.ShapeDtypeStruct((N, 1), _f32)),
        x, W1, degp)

    acc1 = _row_pass(h1hat, src2g, dst2g, HID, GCH, 32, 9, 1)

    h2p, a_s, a_d, mval = _tc(
        _tck2,
        (jax.ShapeDtypeStruct((N, HID), _f32),
         jax.ShapeDtypeStruct((N, 1), _f32),
         jax.ShapeDtypeStruct((N, 1), _f32),
         jax.ShapeDtypeStruct((1, 1), _f32)),
        acc1, h1hat, dinv, Wg, att_src.reshape(HID, 1),
        att_dst.reshape(HID, 1), b1.reshape(1, HID))

    asv = jnp.pad(a_s.reshape(-1), (0, NPT - N))
    adv = jnp.pad(a_d.reshape(-1), (0, NPT - N))
    mvec = jnp.broadcast_to(mval.reshape(1), (16,))

    wgt, denp = _wgt_pass(src2d, dst2d, asv, adv, mvec)
    w2g = wgt.reshape(E_PAD // GCH, GCH)

    nump = _row_pass(h2p, src2g, dst2g, HID, GCH, 40, 7, 1, wgt2d=w2g)

    h3hat = _tc(
        _tck3,
        jax.ShapeDtypeStruct((N, LBL), _f32),
        nump, denp, a_s, a_d, mval, h2p, dinv, W2, bg.reshape(1, HID))

    acc3 = _row_pass(h3hat, src2d, dst2d, LBL, CH, 16, 9, 1)

    y = _tc(
        _tck4,
        jax.ShapeDtypeStruct((N, LBL), _f32),
        acc3, h3hat, dinv, b2.reshape(1, LBL))
    return y


# speedup vs baseline: 1.3209x; 1.0019x over previous
import functools

import jax
import jax.numpy as jnp
from jax import lax
from jax.experimental import pallas as pl
from jax.experimental.pallas import tpu as pltpu
from jax.experimental.pallas import tpu_sc as plsc

N = 10000
E = 320000
HID = 128
LBL = 64

NC, NS = 2, 16
NW = NC * NS
CH = 128
EW = 10240
E_PAD = NW * EW
NCHUNK = EW // CH
GCH = 64
GNCH = EW // GCH
A_ROWS = 10240
RPT = A_ROWS // NS
NPT = 10016

_MESH = dict(core_axis_name="c", subcore_axis_name="s",
             num_cores=NC, num_subcores=NS)
_SC_PARAMS = pltpu.CompilerParams(needs_layout_passes=False,
                                  use_tc_tiling_on_sc=False)

_f32 = jnp.float32
_i32 = jnp.int32


def _fill(ref, n_rows, n_cols, val):
    v = jnp.full((16,), val, _f32)

    def body(i, _):
        for r in range(n_cols // 16):
            ref[i, pl.ds(r * 16, 16)] = v
        return 0

    lax.fori_loop(0, n_rows, body, 0)


def _zero_acc_slice(zbuf, acc, s):
    def body(k, _):
        pltpu.sync_copy(zbuf, acc.at[pl.ds(s * RPT + k * 16, 16)])
        return 0

    lax.fori_loop(0, RPT // 16, body, 0)


def _row_pass(table, src2d, dst2d, d, ch, seg, nseg0, nseg1, wgt2d=None):
    NB = 4
    assert seg % NB == 0
    n0, n1 = seg * nseg0, seg * nseg1
    assert NS * (n0 + n1) == E_PAD // ch
    weighted = wgt2d is not None

    scratch = [
        pltpu.VMEM((seg, ch), _i32),
        pltpu.VMEM((seg, ch), _i32),
        pltpu.VMEM((NB, ch, d), _f32),
        pltpu.VMEM((16, d), _f32),
        pltpu.VMEM_SHARED((A_ROWS, d), _f32),
        pltpu.SemaphoreType.DMA((NB,)),
    ]
    if weighted:
        scratch.insert(2, pltpu.VMEM((seg, ch), _f32))

    @functools.partial(
        pl.kernel,
        out_type=jax.ShapeDtypeStruct((NC, N, d), _f32),
        mesh=plsc.VectorSubcoreMesh(**_MESH),
        compiler_params=_SC_PARAMS,
        scratch_types=scratch,
    )
    def body(*refs):
        if weighted:
            (tab, src, dst, wgt, out,
             idx_sb, idx_db, wb, rows, zbuf, acc, gsem) = refs
        else:
            (tab, src, dst, out,
             idx_sb, idx_db, rows, zbuf, acc, gsem) = refs
        c = lax.axis_index("c")
        s = lax.axis_index("s")
        _fill(zbuf, 16, d, 0.0)
        _zero_acc_slice(zbuf, acc, s)
        plsc.subcore_barrier()
        nseg_c = jnp.where(c == 0, nseg0, nseg1)
        tbase = jnp.where(c == 0, s * n0, NS * n0 + s * n1)

        for h in range(max(nseg0, nseg1)):
            @pl.when(h < nseg_c)
            def _():
                srow = tbase + h * seg
                pltpu.sync_copy(src.at[pl.ds(srow, seg)], idx_sb)
                pltpu.sync_copy(dst.at[pl.ds(srow, seg)], idx_db)
                if weighted:
                    pltpu.sync_copy(wgt.at[pl.ds(srow, seg)], wb)
                for p in range(NB - 1):
                    @pl.when(c == 1)
                    def _(p=p):
                        pltpu.async_copy(
                            tab.at[idx_sb.at[p]], rows.at[p], gsem.at[p],
                            priority=1)

                    @pl.when(c == 0)
                    def _(p=p):
                        pltpu.async_copy(
                            tab.at[idx_sb.at[p]], rows.at[p], gsem.at[p])

                def chunk(i, _):
                    b = lax.rem(i, NB)
                    pltpu.make_async_copy(
                        tab.at[idx_sb.at[i]], rows.at[b], gsem.at[b]).wait()
                    nxt = i + NB - 1
                    nb_ = lax.rem(nxt, NB)

                    @pl.when((nxt < seg) & (c == 1))
                    def _():
                        pltpu.async_copy(
                            tab.at[idx_sb.at[nxt]], rows.at[nb_],
                            gsem.at[nb_], priority=1)

                    @pl.when((nxt < seg) & (c == 0))
                    def _():
                        pltpu.async_copy(
                            tab.at[idx_sb.at[nxt]], rows.at[nb_],
                            gsem.at[nb_])

                    if weighted:
                        for g2 in range(ch // 16):
                            w16 = wb[i, pl.ds(g2 * 16, 16)]
                            for j in range(16):
                                w = w16[j]
                                row = g2 * 16 + j
                                for r in range(d // 16):
                                    rows[b, row, pl.ds(r * 16, 16)] = (
                                        rows[b, row, pl.ds(r * 16, 16)] * w)
                    pltpu.sync_copy(rows.at[b], acc.at[idx_db.at[i]], add=True)
                    return 0

                lax.fori_loop(0, seg, chunk, 0)

        plsc.subcore_barrier()

        @pl.when(s == 0)
        def _():
            pltpu.sync_copy(acc.at[pl.ds(0, N)], out.at[c])

    if weighted:
        return body(table, src2d, dst2d, wgt2d)
    return body(table, src2d, dst2d)


def _deg_pass(dstp):
    @functools.partial(
        pl.kernel,
        out_type=jax.ShapeDtypeStruct((NC, N, 16), _f32),
        mesh=plsc.VectorSubcoreMesh(**_MESH),
        compiler_params=_SC_PARAMS,
        scratch_types=[
            pltpu.VMEM((CH,), _i32),
            pltpu.VMEM((CH, 16), _f32),
            pltpu.VMEM((16, 16), _f32),
            pltpu.VMEM_SHARED((A_ROWS, 16), _f32),
        ],
    )
    def body(dst, out, idx_d, crow, zbuf, acc):
        c = lax.axis_index("c")
        s = lax.axis_index("s")
        _fill(zbuf, 16, 16, 0.0)
        _fill(crow, CH, 16, 1.0)
        _zero_acc_slice(zbuf, acc, s)
        plsc.subcore_barrier()
        base0 = (c * NS + s) * EW

        def chunk(i, _):
            base = base0 + i * CH
            pltpu.sync_copy(dst.at[pl.ds(base, CH)], idx_d)
            pltpu.sync_copy(crow, acc.at[idx_d], add=True)
            return 0

        lax.fori_loop(0, NCHUNK, chunk, 0)
        plsc.subcore_barrier()

        @pl.when(s == 0)
        def _():
            pltpu.sync_copy(acc.at[pl.ds(0, N)], out.at[c])

    return body(dstp)


def _wgt_pass(src2d, dst2d, asv, adv, mvec):
    @functools.partial(
        pl.kernel,
        out_type=(
            jax.ShapeDtypeStruct((E_PAD // CH, CH), _f32),
            jax.ShapeDtypeStruct((NC, N, 16), _f32),
        ),
        mesh=plsc.VectorSubcoreMesh(**_MESH),
        compiler_params=_SC_PARAMS,
        scratch_types=[
            pltpu.VMEM((NCHUNK, CH), _i32),
            pltpu.VMEM((NCHUNK, CH), _i32),
            pltpu.VMEM((NCHUNK, CH), _f32),
            pltpu.VMEM((CH, 16), _f32),
            pltpu.VMEM((NPT,), _f32),
            pltpu.VMEM((NPT,), _f32),
            pltpu.VMEM((16,), _f32),
            pltpu.VMEM((16, 16), _f32),
            pltpu.VMEM_SHARED((A_ROWS, 16), _f32),
        ],
    )
    def body(src, dst, a_s, a_d, mv, wout, dout,
             idx_sb, idx_db, wb, ex16, asb, adb, mvb, zb16, den):
        c = lax.axis_index("c")
        s = lax.axis_index("s")
        wrow = (c * NS + s) * NCHUNK
        pltpu.sync_copy(a_s, asb)
        pltpu.sync_copy(a_d, adb)
        pltpu.sync_copy(mv, mvb)
        pltpu.sync_copy(src.at[pl.ds(wrow, NCHUNK)], idx_sb)
        pltpu.sync_copy(dst.at[pl.ds(wrow, NCHUNK)], idx_db)
        _fill(zb16, 16, 16, 0.0)
        _fill(ex16, CH, 16, 0.0)
        _zero_acc_slice(zb16, den, s)
        plsc.subcore_barrier()
        mvv = mvb[pl.ds(0, 16)]
        czero = jnp.zeros((16,), _i32)
        ar16 = jnp.arange(16, dtype=_i32)

        def chunk(i, _):
            for g in range(CH // 16):
                s16 = idx_sb[i, pl.ds(g * 16, 16)]
                d16 = idx_db[i, pl.ds(g * 16, 16)]
                e = plsc.load_gather(asb, [s16]) + plsc.load_gather(adb, [d16])
                e = jnp.where(e >= 0.0, e, e * 0.2)
                ex = jnp.exp(e - mvv)
                wb[i, pl.ds(g * 16, 16)] = ex
                plsc.store_scatter(ex16, [g * 16 + ar16, czero], ex)
            pltpu.sync_copy(ex16, den.at[idx_db.at[i]], add=True)
            return 0

        lax.fori_loop(0, NCHUNK, chunk, 0)
        pltpu.sync_copy(wb, wout.at[pl.ds(wrow, NCHUNK)])
        plsc.subcore_barrier()

        @pl.when(s == 0)
        def _():
            pltpu.sync_copy(den.at[pl.ds(0, N)], dout.at[c])

    return body(src2d, dst2d, asv, adv, mvec)


def _tck1(x_ref, w1_ref, degp_ref, h1hat_ref, dinv_ref):
    x = x_ref[...]
    mu = jnp.mean(x, axis=0, keepdims=True)
    xc = x - mu
    sd = jnp.sqrt(jnp.sum(xc * xc, axis=0, keepdims=True) * (1.0 / (N - 1)))
    xn = xc / sd
    h1p = jnp.dot(xn, w1_ref[...], preferred_element_type=_f32)
    dp = degp_ref[...]
    deg = dp[0, :, 0:1] + dp[1, :, 0:1] + 1.0
    dinv = lax.rsqrt(deg)
    dinv_ref[...] = dinv
    h1hat_ref[...] = dinv * h1p


def _tck2(accp_ref, h1hat_ref, dinv_ref, wg_ref, atts_ref, attd_ref,
          b1_ref, h2p_ref, as_ref, ad_ref, m_ref):
    a = accp_ref[...]
    dinv = dinv_ref[...]
    out1 = dinv * (a[0] + a[1] + h1hat_ref[...]) + b1_ref[...]
    h1 = jnp.maximum(out1, 0.0)
    h2p = jnp.dot(h1, wg_ref[...], preferred_element_type=_f32)
    a_s = jnp.dot(h2p, atts_ref[...], preferred_element_type=_f32)
    a_d = jnp.dot(h2p, attd_ref[...], preferred_element_type=_f32)
    m = jnp.max(a_s) + jnp.max(a_d)
    m = jnp.where(m >= 0.0, m, m * 0.2)
    h2p_ref[...] = h2p
    as_ref[...] = a_s
    ad_ref[...] = a_d
    m_ref[...] = jnp.reshape(m, (1, 1))


def _tck3(nump_ref, denp_ref, as_ref, ad_ref, m_ref, h2p_ref, dinv_ref,
          w2_ref, bg_ref, h3hat_ref):
    m = m_ref[0, 0]
    es = as_ref[...] + ad_ref[...]
    es = jnp.where(es >= 0.0, es, es * 0.2)
    exs = jnp.exp(es - m)
    npv = nump_ref[...]
    h2p = h2p_ref[...]
    num = npv[0] + npv[1] + exs * h2p
    dp = denp_ref[...]
    den = dp[0, :, 0:1] + dp[1, :, 0:1] + exs
    h2 = jnp.maximum(num / (den + 1e-16) + bg_ref[...], 0.0)
    h3p = jnp.dot(h2, w2_ref[...], preferred_element_type=_f32)
    h3hat_ref[...] = dinv_ref[...] * h3p


def _tck4(accp_ref, h3hat_ref, dinv_ref, b2_ref, y_ref):
    a = accp_ref[...]
    out3 = dinv_ref[...] * (a[0] + a[1] + h3hat_ref[...]) + b2_ref[...]
    mx = jnp.max(out3, axis=1, keepdims=True)
    lse = mx + jnp.log(jnp.sum(jnp.exp(out3 - mx), axis=1, keepdims=True))
    y_ref[...] = out3 - lse


def _tc(body, out_shape, *args):
    return pl.pallas_call(body, out_shape=out_shape)(*args)


def kernel(x, edge_index, W1, b1, Wg, att_src, att_dst, bg, W2, b2):
    src = edge_index[0].astype(_i32)
    dst = edge_index[1].astype(_i32)
    pad = E_PAD - E
    srcp = jnp.concatenate([src, jnp.zeros((pad,), _i32)])
    dstp = jnp.concatenate([dst, jnp.full((pad,), N, _i32)])
    src2d = srcp.reshape(E_PAD // CH, CH)
    dst2d = dstp.reshape(E_PAD // CH, CH)
    src2g = srcp.reshape(E_PAD // GCH, GCH)
    dst2g = dstp.reshape(E_PAD // GCH, GCH)

    degp = _deg_pass(dstp)

    h1hat, dinv = _tc(
        _tck1,
        (jax.ShapeDtypeStruct((N, HID), _f32),
         jax.ShapeDtypeStruct((N, 1), _f32)),
        x, W1, degp)

    acc1 = _row_pass(h1hat, src2g, dst2g, HID, GCH, 32, 9, 1)

    h2p, a_s, a_d, mval = _tc(
        _tck2,
        (jax.ShapeDtypeStruct((N, HID), _f32),
         jax.ShapeDtypeStruct((N, 1), _f32),
         jax.ShapeDtypeStruct((N, 1), _f32),
         jax.ShapeDtypeStruct((1, 1), _f32)),
        acc1, h1hat, dinv, Wg, att_src.reshape(HID, 1),
        att_dst.reshape(HID, 1), b1.reshape(1, HID))

    asv = jnp.pad(a_s.reshape(-1), (0, NPT - N))
    adv = jnp.pad(a_d.reshape(-1), (0, NPT - N))
    mvec = jnp.broadcast_to(mval.reshape(1), (16,))

    wgt, denp = _wgt_pass(src2d, dst2d, asv, adv, mvec)
    w2g = wgt.reshape(E_PAD // GCH, GCH)

    nump = _row_pass(h2p, src2g, dst2g, HID, GCH, 40, 7, 1, wgt2d=w2g)

    h3hat = _tc(
        _tck3,
        jax.ShapeDtypeStruct((N, LBL), _f32),
        nump, denp, a_s, a_d, mval, h2p, dinv, W2, bg.reshape(1, HID))

    acc3 = _row_pass(h3hat, src2d, dst2d, LBL, CH, 16, 9, 1)

    y = _tc(
        _tck4,
        jax.ShapeDtypeStruct((N, LBL), _f32),
        acc3, h3hat, dinv, b2.reshape(1, LBL))
    return y
